# baseline (speedup 1.0000x reference)
import functools

import jax
import jax.numpy as jnp
from jax import lax
from jax.experimental import pallas as pl
from jax.experimental.pallas import tpu as pltpu
from jax.experimental.pallas import tpu_sc as plsc

NC = 2
NS = 16
NW = NC * NS
EPB = 256
LANES = 16
BN_EPS = 1e-5
JUNK = 1024


def _segment_sum_sc(n_nodes, feat, units_per_tile):
    rows_per_tile = n_nodes // NS
    n_acc = n_nodes + JUNK
    upt = units_per_tile
    mesh = plsc.VectorSubcoreMesh(
        core_axis_name="c", subcore_axis_name="s", num_cores=NC, num_subcores=NS
    )

    @functools.partial(
        pl.kernel,
        out_type=jax.ShapeDtypeStruct((n_nodes, NC * feat), jnp.float32),
        mesh=mesh,
        scratch_types=[
            pltpu.VMEM((upt, EPB), jnp.int32),
            pltpu.VMEM((upt, EPB), jnp.int32),
            pltpu.VMEM((3 * EPB, feat), jnp.float32),
            pltpu.VMEM((160, feat), jnp.float32),
            pltpu.VMEM_SHARED((n_acc, feat), jnp.float32),
            pltpu.SemaphoreType.DMA,
            pltpu.SemaphoreType.DMA,
        ],
        compiler_params=pltpu.CompilerParams(
            use_tc_tiling_on_sc=False, skip_device_barrier=True
        ),
    )
    def segsum(y_hbm, src_hbm, dst_hbm, out_hbm,
               src_v, dst_v, rows_v, stage_v, acc_sh, sem_g, sem_s):
        c = lax.axis_index("c")
        s = lax.axis_index("s")
        wid = c * NS + s

        zv = jnp.zeros((LANES,), jnp.float32)
        chunks = []
        left = rows_per_tile
        while left > 0:
            chunks.append(min(160, left))
            left -= chunks[-1]

        def zero_row(i, carry):
            for g in range(feat // LANES):
                stage_v[i, pl.ds(g * LANES, LANES)] = zv
            return carry

        lax.fori_loop(0, chunks[0], zero_row, 0)
        base = s * rows_per_tile
        off = 0
        for ch in chunks:
            pltpu.sync_copy(
                stage_v.at[pl.ds(0, ch)], acc_sh.at[pl.ds(base + off, ch)]
            )
            off += ch

        pltpu.sync_copy(src_hbm.at[pl.ds(wid * upt, upt)], src_v)
        pltpu.sync_copy(dst_hbm.at[pl.ds(wid * upt, upt)], dst_v)

        plsc.subcore_barrier()

        for w in range(2):
            pltpu.async_copy(
                y_hbm.at[src_v.at[w]], rows_v.at[pl.ds(w * EPB, EPB)], sem_g
            )

        def body(u, carry):
            slot = lax.rem(u, 3) * EPB
            pltpu.make_async_copy(
                y_hbm.at[src_v.at[u]], rows_v.at[pl.ds(slot, EPB)], sem_g
            ).wait()
            pltpu.async_copy(
                rows_v.at[pl.ds(slot, EPB)], acc_sh.at[dst_v.at[u]], sem_s, add=True
            )

            @pl.when(u >= 1)
            def _():
                oslot = lax.rem(u + 2, 3) * EPB
                pltpu.make_async_copy(
                    rows_v.at[pl.ds(oslot, EPB)], acc_sh.at[dst_v.at[u - 1]], sem_s
                ).wait()

            @pl.when(u + 2 < upt)
            def _():
                nslot = lax.rem(u + 2, 3) * EPB
                pltpu.async_copy(
                    y_hbm.at[src_v.at[u + 2]], rows_v.at[pl.ds(nslot, EPB)], sem_g
                )

            return carry

        lax.fori_loop(0, upt, body, 0)
        pltpu.make_async_copy(
            rows_v.at[pl.ds(0, EPB)], acc_sh.at[dst_v.at[upt - 1]], sem_s
        ).wait()

        plsc.subcore_barrier()

        off = 0
        for ch in chunks:
            pltpu.sync_copy(
                acc_sh.at[pl.ds(base + off, ch)], stage_v.at[pl.ds(0, ch)]
            )
            pltpu.sync_copy(
                stage_v.at[pl.ds(0, ch)],
                out_hbm.at[pl.ds(base + off, ch), pl.ds(c * feat, feat)],
            )
            off += ch

    return segsum


def _first_linear(n_nodes, feat):
    def body(x_ref, w_ref, o_ref):
        o_ref[:, pl.ds(0, feat)] = jnp.dot(
            x_ref[...], w_ref[...], preferred_element_type=jnp.float32
        )

    return pl.pallas_call(
        body, out_shape=jax.ShapeDtypeStruct((n_nodes, 2 * feat), jnp.float32)
    )


def _mlp_tail_next(n_nodes, feat):
    def body(y_ref, p_ref, b1_ref, w2t_ref, b2_ref, g_ref, be_ref, w1tn_ref, o_ref):
        t = (
            y_ref[:, pl.ds(0, feat)]
            + p_ref[:, pl.ds(0, feat)]
            + p_ref[:, pl.ds(feat, feat)]
            + b1_ref[...]
        )
        t = jnp.maximum(t, 0.0)
        u = jnp.dot(t, w2t_ref[...], preferred_element_type=jnp.float32) + b2_ref[...]
        u = jnp.maximum(u, 0.0)
        mu = jnp.mean(u, axis=0, keepdims=True)
        var = jnp.mean((u - mu) ** 2, axis=0, keepdims=True)
        hn = (u - mu) * jax.lax.rsqrt(var + BN_EPS) * g_ref[...] + be_ref[...]
        o_ref[:, pl.ds(0, feat)] = jnp.dot(
            hn, w1tn_ref[...], preferred_element_type=jnp.float32
        )

    return pl.pallas_call(
        body, out_shape=jax.ShapeDtypeStruct((n_nodes, 2 * feat), jnp.float32)
    )


def _mlp_tail_head(n_nodes, feat):
    def body(y_ref, p_ref, b1_ref, w2t_ref, b2_ref, g_ref, be_ref,
             wot_ref, bo_ref, o_ref):
        t = (
            y_ref[:, pl.ds(0, feat)]
            + p_ref[:, pl.ds(0, feat)]
            + p_ref[:, pl.ds(feat, feat)]
            + b1_ref[...]
        )
        t = jnp.maximum(t, 0.0)
        u = jnp.dot(t, w2t_ref[...], preferred_element_type=jnp.float32) + b2_ref[...]
        u = jnp.maximum(u, 0.0)
        mu = jnp.mean(u, axis=0, keepdims=True)
        var = jnp.mean((u - mu) ** 2, axis=0, keepdims=True)
        hn = (u - mu) * jax.lax.rsqrt(var + BN_EPS) * g_ref[...] + be_ref[...]
        pooled = jnp.mean(hn, axis=0, keepdims=True)
        logit = jnp.dot(pooled, wot_ref[...], preferred_element_type=jnp.float32)
        o_ref[...] = jax.nn.sigmoid(logit + bo_ref[...])

    return pl.pallas_call(
        body, out_shape=jax.ShapeDtypeStruct((1, 1), jnp.float32)
    )


def kernel(x, edge_index, batch, params):
    n_nodes, d_in, m = x.shape
    n_edges = edge_index.shape[1]
    layers = params["layers"]
    feat = layers[0][0].shape[0]
    upt = -(-n_edges // (NW * EPB * 8)) * 8
    e_pad = NW * EPB * upt
    pad_src = jnp.arange(e_pad - n_edges, dtype=jnp.int32) % n_nodes
    src = (jnp.concatenate([edge_index[0], pad_src]) * 2).reshape(NW * upt, EPB)
    pad_dst = n_nodes + jnp.arange(e_pad - n_edges, dtype=jnp.int32) % JUNK
    dst = jnp.concatenate([edge_index[1], pad_dst]).reshape(NW * upt, EPB)

    segsum = _segment_sum_sc(n_nodes, feat, upt)
    first = _first_linear(n_nodes, feat)
    mid = _mlp_tail_next(n_nodes, feat)
    head = _mlp_tail_head(n_nodes, feat)

    x2d = x[:, :, 0]
    y = first(x2d, layers[0][0].T)
    out = None
    for i, (w1, b1, w2, b2, gamma, beta) in enumerate(layers):
        p = segsum(y.reshape(2 * n_nodes, feat), src, dst)
        b1r = b1.reshape(1, feat)
        b2r = b2.reshape(1, feat)
        gr = gamma.reshape(1, feat)
        ber = beta.reshape(1, feat)
        if i + 1 < len(layers):
            y = mid(y, p, b1r, w2.T, b2r, gr, ber, layers[i + 1][0].T)
        else:
            out = head(
                y, p, b1r, w2.T, b2r, gr, ber,
                params["wout"].T, params["bout"].reshape(1, 1),
            )
    return out.reshape(1, m, 1)


# speedup vs baseline: 1.0038x; 1.0038x over previous
import functools

import jax
import jax.numpy as jnp
from jax import lax
from jax.experimental import pallas as pl
from jax.experimental.pallas import tpu as pltpu
from jax.experimental.pallas import tpu_sc as plsc

NC = 2
NS = 16
NW = NC * NS
EPB = 128
LANES = 16
BN_EPS = 1e-5
JUNK = 1024


def _segment_sum_sc(n_nodes, feat, units_per_tile):
    rows_per_tile = n_nodes // NS
    n_acc = n_nodes + JUNK
    upt = units_per_tile
    mesh = plsc.VectorSubcoreMesh(
        core_axis_name="c", subcore_axis_name="s", num_cores=NC, num_subcores=NS
    )

    @functools.partial(
        pl.kernel,
        out_type=jax.ShapeDtypeStruct((n_nodes, NC * feat), jnp.float32),
        mesh=mesh,
        scratch_types=[
            pltpu.VMEM((upt, EPB), jnp.int32),
            pltpu.VMEM((upt, EPB), jnp.int32),
            pltpu.VMEM((5 * EPB, feat), jnp.float32),
            pltpu.VMEM((160, feat), jnp.float32),
            pltpu.VMEM_SHARED((n_acc, feat), jnp.float32),
            pltpu.SemaphoreType.DMA,
            pltpu.SemaphoreType.DMA,
        ],
        compiler_params=pltpu.CompilerParams(
            use_tc_tiling_on_sc=False, skip_device_barrier=True
        ),
    )
    def segsum(y_hbm, src_hbm, dst_hbm, out_hbm,
               src_v, dst_v, rows_v, stage_v, acc_sh, sem_g, sem_s):
        c = lax.axis_index("c")
        s = lax.axis_index("s")
        wid = c * NS + s

        zv = jnp.zeros((LANES,), jnp.float32)
        chunks = []
        left = rows_per_tile
        while left > 0:
            chunks.append(min(160, left))
            left -= chunks[-1]

        def zero_row(i, carry):
            for g in range(feat // LANES):
                stage_v[i, pl.ds(g * LANES, LANES)] = zv
            return carry

        lax.fori_loop(0, chunks[0], zero_row, 0)
        base = s * rows_per_tile
        off = 0
        for ch in chunks:
            pltpu.sync_copy(
                stage_v.at[pl.ds(0, ch)], acc_sh.at[pl.ds(base + off, ch)]
            )
            off += ch

        pltpu.sync_copy(src_hbm.at[pl.ds(wid * upt, upt)], src_v)
        pltpu.sync_copy(dst_hbm.at[pl.ds(wid * upt, upt)], dst_v)

        plsc.subcore_barrier()

        for w in range(4):
            pltpu.async_copy(
                y_hbm.at[src_v.at[w]], rows_v.at[pl.ds(w * EPB, EPB)], sem_g
            )

        def body(u, carry):
            slot = lax.rem(u, 5) * EPB
            pltpu.make_async_copy(
                y_hbm.at[src_v.at[u]], rows_v.at[pl.ds(slot, EPB)], sem_g
            ).wait()
            pltpu.async_copy(
                rows_v.at[pl.ds(slot, EPB)], acc_sh.at[dst_v.at[u]], sem_s, add=True
            )

            @pl.when(u >= 1)
            def _():
                oslot = lax.rem(u + 4, 5) * EPB
                pltpu.make_async_copy(
                    rows_v.at[pl.ds(oslot, EPB)], acc_sh.at[dst_v.at[u - 1]], sem_s
                ).wait()

            @pl.when(u + 4 < upt)
            def _():
                nslot = lax.rem(u + 4, 5) * EPB
                pltpu.async_copy(
                    y_hbm.at[src_v.at[u + 4]], rows_v.at[pl.ds(nslot, EPB)], sem_g
                )

            return carry

        lax.fori_loop(0, upt, body, 0)
        pltpu.make_async_copy(
            rows_v.at[pl.ds(0, EPB)], acc_sh.at[dst_v.at[upt - 1]], sem_s
        ).wait()

        plsc.subcore_barrier()

        off = 0
        for ch in chunks:
            pltpu.sync_copy(
                acc_sh.at[pl.ds(base + off, ch)], stage_v.at[pl.ds(0, ch)]
            )
            pltpu.sync_copy(
                stage_v.at[pl.ds(0, ch)],
                out_hbm.at[pl.ds(base + off, ch), pl.ds(c * feat, feat)],
            )
            off += ch

    return segsum


def _first_linear(n_nodes, feat):
    def body(x_ref, w_ref, o_ref):
        o_ref[:, pl.ds(0, feat)] = jnp.dot(
            x_ref[...], w_ref[...], preferred_element_type=jnp.float32
        )

    return pl.pallas_call(
        body, out_shape=jax.ShapeDtypeStruct((n_nodes, 2 * feat), jnp.float32)
    )


def _mlp_tail_next(n_nodes, feat):
    def body(y_ref, p_ref, b1_ref, w2t_ref, b2_ref, g_ref, be_ref, w1tn_ref, o_ref):
        t = (
            y_ref[:, pl.ds(0, feat)]
            + p_ref[:, pl.ds(0, feat)]
            + p_ref[:, pl.ds(feat, feat)]
            + b1_ref[...]
        )
        t = jnp.maximum(t, 0.0)
        u = jnp.dot(t, w2t_ref[...], preferred_element_type=jnp.float32) + b2_ref[...]
        u = jnp.maximum(u, 0.0)
        mu = jnp.mean(u, axis=0, keepdims=True)
        var = jnp.mean((u - mu) ** 2, axis=0, keepdims=True)
        hn = (u - mu) * jax.lax.rsqrt(var + BN_EPS) * g_ref[...] + be_ref[...]
        o_ref[:, pl.ds(0, feat)] = jnp.dot(
            hn, w1tn_ref[...], preferred_element_type=jnp.float32
        )

    return pl.pallas_call(
        body, out_shape=jax.ShapeDtypeStruct((n_nodes, 2 * feat), jnp.float32)
    )


def _mlp_tail_head(n_nodes, feat):
    def body(y_ref, p_ref, b1_ref, w2t_ref, b2_ref, g_ref, be_ref,
             wot_ref, bo_ref, o_ref):
        t = (
            y_ref[:, pl.ds(0, feat)]
            + p_ref[:, pl.ds(0, feat)]
            + p_ref[:, pl.ds(feat, feat)]
            + b1_ref[...]
        )
        t = jnp.maximum(t, 0.0)
        u = jnp.dot(t, w2t_ref[...], preferred_element_type=jnp.float32) + b2_ref[...]
        u = jnp.maximum(u, 0.0)
        mu = jnp.mean(u, axis=0, keepdims=True)
        var = jnp.mean((u - mu) ** 2, axis=0, keepdims=True)
        hn = (u - mu) * jax.lax.rsqrt(var + BN_EPS) * g_ref[...] + be_ref[...]
        pooled = jnp.mean(hn, axis=0, keepdims=True)
        logit = jnp.dot(pooled, wot_ref[...], preferred_element_type=jnp.float32)
        o_ref[...] = jax.nn.sigmoid(logit + bo_ref[...])

    return pl.pallas_call(
        body, out_shape=jax.ShapeDtypeStruct((1, 1), jnp.float32)
    )


def kernel(x, edge_index, batch, params):
    n_nodes, d_in, m = x.shape
    n_edges = edge_index.shape[1]
    layers = params["layers"]
    feat = layers[0][0].shape[0]
    upt = -(-n_edges // (NW * EPB * 8)) * 8
    e_pad = NW * EPB * upt
    pad_src = jnp.arange(e_pad - n_edges, dtype=jnp.int32) % n_nodes
    src = (jnp.concatenate([edge_index[0], pad_src]) * 2).reshape(NW * upt, EPB)
    pad_dst = n_nodes + jnp.arange(e_pad - n_edges, dtype=jnp.int32) % JUNK
    dst = jnp.concatenate([edge_index[1], pad_dst]).reshape(NW * upt, EPB)

    segsum = _segment_sum_sc(n_nodes, feat, upt)
    first = _first_linear(n_nodes, feat)
    mid = _mlp_tail_next(n_nodes, feat)
    head = _mlp_tail_head(n_nodes, feat)

    x2d = x[:, :, 0]
    y = first(x2d, layers[0][0].T)
    out = None
    for i, (w1, b1, w2, b2, gamma, beta) in enumerate(layers):
        p = segsum(y.reshape(2 * n_nodes, feat), src, dst)
        b1r = b1.reshape(1, feat)
        b2r = b2.reshape(1, feat)
        gr = gamma.reshape(1, feat)
        ber = beta.reshape(1, feat)
        if i + 1 < len(layers):
            y = mid(y, p, b1r, w2.T, b2r, gr, ber, layers[i + 1][0].T)
        else:
            out = head(
                y, p, b1r, w2.T, b2r, gr, ber,
                params["wout"].T, params["bout"].reshape(1, 1),
            )
    return out.reshape(1, m, 1)
